# Initial kernel scaffold; baseline (speedup 1.0000x reference)
import jax
import jax.numpy as jnp
from jax.experimental import pallas as pl


def kernel(atomic_numbers, mask_weight):
    raise NotImplementedError("write your pallas kernel here")


# speedup vs baseline: 4.3478x; 4.3478x over previous
import functools

import jax
import jax.numpy as jnp
from jax import lax
from jax.experimental import pallas as pl
from jax.experimental.pallas import tpu as pltpu
from jax.experimental.pallas import tpu_sc as plsc

NUM_WORKERS = 32
CHUNK = 5120
TABLE_PAD = 1024


def _build_sc_gather(M, D):
    b_per_w = M // NUM_WORKERS
    n_iters = b_per_w // CHUNK
    groups = CHUNK // 16
    mesh = plsc.VectorSubcoreMesh(core_axis_name="c", subcore_axis_name="s")

    @functools.partial(
        pl.kernel,
        mesh=mesh,
        out_type=jax.ShapeDtypeStruct((M * D,), jnp.float32),
        compiler_params=pltpu.CompilerParams(needs_layout_passes=False),
        scratch_types=[
            pltpu.VMEM((TABLE_PAD,), jnp.float32),
            pltpu.VMEM((CHUNK,), jnp.int32),
            pltpu.VMEM((CHUNK * D,), jnp.float32),
        ],
    )
    def gather_kernel(table_hbm, idx_hbm, out_hbm, table_v, idx_v, rows_v):
        wid = lax.axis_index("s") * 2 + lax.axis_index("c")
        base = wid * b_per_w
        pltpu.sync_copy(table_hbm, table_v)
        lane = lax.iota(jnp.int32, 16)
        lane_d = lane * D

        def chunk_body(it, carry):
            off = base + it * CHUNK
            pltpu.sync_copy(idx_hbm.at[pl.ds(off, CHUNK)], idx_v)

            def group_body(g, c2):
                rows = idx_v[pl.ds(g * 16, 16)]
                flat = rows * D
                gbase = g * (16 * D)
                for j in range(D):
                    vals = plsc.load_gather(table_v, [flat + j])
                    plsc.store_scatter(rows_v, [lane_d + (gbase + j)], vals)
                return c2

            lax.fori_loop(0, groups, group_body, 0)
            pltpu.sync_copy(rows_v, out_hbm.at[pl.ds(off * D, CHUNK * D)])
            return carry

        lax.fori_loop(0, n_iters, chunk_body, 0)

    return gather_kernel


def kernel(atomic_numbers, mask_weight):
    B, L = atomic_numbers.shape
    V, D = mask_weight.shape
    M = B * L
    idx_flat = atomic_numbers.reshape(M)
    table_flat = jnp.zeros((TABLE_PAD,), jnp.float32).at[: V * D].set(
        mask_weight.reshape(V * D)
    )
    out = _build_sc_gather(M, D)(table_flat, idx_flat)
    return out.reshape(B, L, D)


# speedup vs baseline: 6.0847x; 1.3995x over previous
import functools

import jax
import jax.numpy as jnp
from jax import lax
from jax.experimental import pallas as pl
from jax.experimental.pallas import tpu as pltpu
from jax.experimental.pallas import tpu_sc as plsc

NUM_WORKERS = 32
CHUNK = 800
TABLE_PAD = 1024


def _build_sc_gather(M, D):
    b_per_w = M // NUM_WORKERS
    n_iters = b_per_w // CHUNK
    groups = CHUNK // 16
    mesh = plsc.VectorSubcoreMesh(core_axis_name="c", subcore_axis_name="s")

    @functools.partial(
        pl.kernel,
        mesh=mesh,
        out_type=jax.ShapeDtypeStruct((M, D), jnp.float32),
        compiler_params=pltpu.CompilerParams(needs_layout_passes=False),
        scratch_types=[
            pltpu.VMEM((TABLE_PAD,), jnp.float32),
            pltpu.VMEM((CHUNK,), jnp.int32),
            pltpu.VMEM((CHUNK, D), jnp.float32),
        ],
    )
    def gather_kernel(table_hbm, idx_hbm, out_hbm, table_v, idx_v, rows_v):
        wid = lax.axis_index("s") * 2 + lax.axis_index("c")
        base = wid * b_per_w
        pltpu.sync_copy(table_hbm, table_v)
        lane = lax.iota(jnp.int32, 16)

        def chunk_body(it, carry):
            off = base + it * CHUNK
            pltpu.sync_copy(idx_hbm.at[pl.ds(off, CHUNK)], idx_v)

            def group_body(g, c2):
                rows = idx_v[pl.ds(g * 16, 16)]
                flat = rows * D
                rvec = lane + g * 16
                for j in range(D):
                    vals = plsc.load_gather(table_v, [flat + j])
                    plsc.store_scatter(rows_v, [rvec, jnp.full((16,), j, jnp.int32)], vals)
                return c2

            lax.fori_loop(0, groups, group_body, 0)
            pltpu.sync_copy(rows_v, out_hbm.at[pl.ds(off, CHUNK)])
            return carry

        lax.fori_loop(0, n_iters, chunk_body, 0)

    return gather_kernel


def kernel(atomic_numbers, mask_weight):
    B, L = atomic_numbers.shape
    V, D = mask_weight.shape
    M = B * L
    idx_flat = atomic_numbers.reshape(M)
    table_flat = jnp.zeros((TABLE_PAD,), jnp.float32).at[: V * D].set(
        mask_weight.reshape(V * D)
    )
    out = _build_sc_gather(M, D)(table_flat, idx_flat)
    return out.reshape(B, L, D)
